# initial kernel scaffold (unmeasured)
import jax
import jax.numpy as jnp
from jax import lax
from jax.experimental import pallas as pl
from jax.experimental.pallas import tpu as pltpu

T = 512
D = 1024
V_LOCAL = 8192
V_CHUNK = 2048
N_CHUNKS = V_LOCAL // V_CHUNK
NEG = -1e30


def kernel(x, W, labels):
    def body(x_ref, w_ref, labels_ref, out_ref,
             send_buf, recv_buf, send_sem, recv_sem):
        my_x = lax.axis_index("x")
        my_y = lax.axis_index("y")
        partner = (my_x, 1 - my_y)

        xv = x_ref[...]
        labels_col = labels_ref[...]
        base = my_y * V_LOCAL
        m_run = jnp.full((T, 1), NEG, jnp.float32)
        s_run = jnp.zeros((T, 1), jnp.float32)
        l_run = jnp.full((T, 1), NEG, jnp.float32)
        for c in range(N_CHUNKS):
            logits = jnp.dot(
                xv, w_ref[:, c * V_CHUNK:(c + 1) * V_CHUNK],
                preferred_element_type=jnp.float32,
            )
            cols = lax.broadcasted_iota(jnp.int32, (T, V_CHUNK), 1) \
                + (base + c * V_CHUNK)
            m_c = jnp.max(logits, axis=1, keepdims=True)
            m_new = jnp.maximum(m_run, m_c)
            s_run = s_run * jnp.exp(m_run - m_new) \
                + jnp.sum(jnp.exp(logits - m_new), axis=1, keepdims=True)
            m_run = m_new
            l_c = jnp.max(
                jnp.where(cols == labels_col, logits, NEG),
                axis=1, keepdims=True,
            )
            l_run = jnp.maximum(l_run, l_c)

        send_buf[:, 0:1] = m_run
        send_buf[:, 1:2] = s_run
        send_buf[:, 2:3] = l_run

        barrier = pltpu.get_barrier_semaphore()
        pl.semaphore_signal(
            barrier, inc=1,
            device_id=partner, device_id_type=pl.DeviceIdType.MESH,
        )
        pl.semaphore_wait(barrier, 1)

        rdma = pltpu.make_async_remote_copy(
            src_ref=send_buf,
            dst_ref=recv_buf,
            send_sem=send_sem,
            recv_sem=recv_sem,
            device_id=partner,
            device_id_type=pl.DeviceIdType.MESH,
        )
        rdma.start()
        rdma.wait()

        m1 = recv_buf[:, 0:1]
        s1 = recv_buf[:, 1:2]
        l1 = recv_buf[:, 2:3]
        m = jnp.maximum(m_run, m1)
        s = s_run * jnp.exp(m_run - m) + s1 * jnp.exp(m1 - m)
        lbl = jnp.maximum(l_run, l1)
        out_ref[...] = m + jnp.log(s) - lbl

    out = pl.pallas_call(
        body,
        out_shape=jax.ShapeDtypeStruct((T, 1), jnp.float32),
        in_specs=[
            pl.BlockSpec(memory_space=pltpu.VMEM),
            pl.BlockSpec(memory_space=pltpu.VMEM),
            pl.BlockSpec(memory_space=pltpu.VMEM),
        ],
        out_specs=pl.BlockSpec(memory_space=pltpu.VMEM),
        scratch_shapes=[
            pltpu.VMEM((T, 8), jnp.float32),
            pltpu.VMEM((T, 8), jnp.float32),
            pltpu.SemaphoreType.DMA,
            pltpu.SemaphoreType.DMA,
        ],
        compiler_params=pltpu.CompilerParams(collective_id=0),
    )(x, W, labels.reshape(T, 1))
    return out.reshape(T)


# baseline (device time: 28501 ns/iter reference)
import jax
import jax.numpy as jnp
from jax import lax
from jax.experimental import pallas as pl
from jax.experimental.pallas import tpu as pltpu

T = 512
D = 1024
V_LOCAL = 8192
V_CHUNK = 2048
N_CHUNKS = V_LOCAL // V_CHUNK
NEG = -1e30


def kernel(x, W, labels):
    def body(x_ref, w_ref, labels_ref, out_ref,
             m_run, s_run, l_run, send_buf, recv_buf, send_sem, recv_sem):
        c = pl.program_id(0)
        my_x = lax.axis_index("x")
        my_y = lax.axis_index("y")
        partner = (my_x, 1 - my_y)

        @pl.when(c == 0)
        def _init():
            m_run[...] = jnp.full((T, 1), NEG, jnp.float32)
            s_run[...] = jnp.zeros((T, 1), jnp.float32)
            l_run[...] = jnp.full((T, 1), NEG, jnp.float32)

        logits = jnp.dot(
            x_ref[...], w_ref[...], preferred_element_type=jnp.float32
        )
        cols = lax.broadcasted_iota(jnp.int32, (T, V_CHUNK), 1) \
            + (my_y * V_LOCAL + c * V_CHUNK)
        m_old = m_run[...]
        m_new = jnp.maximum(m_old, jnp.max(logits, axis=1, keepdims=True))
        s_run[...] = s_run[...] * jnp.exp(m_old - m_new) \
            + jnp.sum(jnp.exp(logits - m_new), axis=1, keepdims=True)
        m_run[...] = m_new
        l_c = jnp.max(
            jnp.where(cols == labels_ref[...], logits, NEG),
            axis=1, keepdims=True,
        )
        l_run[...] = jnp.maximum(l_run[...], l_c)

        @pl.when(c == N_CHUNKS - 1)
        def _exchange():
            send_buf[:, 0:1] = m_run[...]
            send_buf[:, 1:2] = s_run[...]
            send_buf[:, 2:3] = l_run[...]

            barrier = pltpu.get_barrier_semaphore()
            pl.semaphore_signal(
                barrier, inc=1,
                device_id=partner, device_id_type=pl.DeviceIdType.MESH,
            )
            pl.semaphore_wait(barrier, 1)

            rdma = pltpu.make_async_remote_copy(
                src_ref=send_buf,
                dst_ref=recv_buf,
                send_sem=send_sem,
                recv_sem=recv_sem,
                device_id=partner,
                device_id_type=pl.DeviceIdType.MESH,
            )
            rdma.start()
            rdma.wait()

            m0 = m_run[...]
            s0 = s_run[...]
            l0 = l_run[...]
            m1 = recv_buf[:, 0:1]
            s1 = recv_buf[:, 1:2]
            l1 = recv_buf[:, 2:3]
            m = jnp.maximum(m0, m1)
            s = s0 * jnp.exp(m0 - m) + s1 * jnp.exp(m1 - m)
            lbl = jnp.maximum(l0, l1)
            out_ref[...] = m + jnp.log(s) - lbl

    out = pl.pallas_call(
        body,
        grid=(N_CHUNKS,),
        out_shape=jax.ShapeDtypeStruct((T, 1), jnp.float32),
        in_specs=[
            pl.BlockSpec((T, D), lambda c: (0, 0)),
            pl.BlockSpec((D, V_CHUNK), lambda c: (0, c)),
            pl.BlockSpec((T, 1), lambda c: (0, 0)),
        ],
        out_specs=pl.BlockSpec((T, 1), lambda c: (0, 0)),
        scratch_shapes=[
            pltpu.VMEM((T, 1), jnp.float32),
            pltpu.VMEM((T, 1), jnp.float32),
            pltpu.VMEM((T, 1), jnp.float32),
            pltpu.VMEM((T, 8), jnp.float32),
            pltpu.VMEM((T, 8), jnp.float32),
            pltpu.SemaphoreType.DMA,
            pltpu.SemaphoreType.DMA,
        ],
        compiler_params=pltpu.CompilerParams(
            collective_id=0,
            dimension_semantics=("arbitrary",),
        ),
    )(x, W, labels.reshape(T, 1).astype(jnp.int32))
    return out.reshape(T)


# device time: 25596 ns/iter; 1.1135x vs baseline; 1.1135x over previous
import jax
import jax.numpy as jnp
from jax import lax
from jax.experimental import pallas as pl
from jax.experimental.pallas import tpu as pltpu

T = 512
D = 1024
V_LOCAL = 8192
V_CHUNK = 2048
N_CHUNKS = V_LOCAL // V_CHUNK
NEG = -1e30


def kernel(x, W, labels):
    def body(x_ref, w_ref, labels_ref, out_ref,
             s_run, l_run, send_buf, recv_buf, send_sem, recv_sem):
        c = pl.program_id(0)
        my_x = lax.axis_index("x")
        my_y = lax.axis_index("y")
        partner = (my_x, 1 - my_y)

        @pl.when(c == 0)
        def _init():
            s_run[...] = jnp.zeros((T, 1), jnp.float32)
            l_run[...] = jnp.zeros((T, 1), jnp.float32)

        logits = jnp.dot(
            x_ref[...], w_ref[...], preferred_element_type=jnp.float32
        )
        cols = lax.broadcasted_iota(jnp.int32, (T, V_CHUNK), 1) \
            + (my_y * V_LOCAL + c * V_CHUNK)
        e = jnp.exp(logits)
        s_run[...] += jnp.sum(e, axis=1, keepdims=True)
        l_run[...] += jnp.sum(
            jnp.where(cols == labels_ref[...], e, 0.0),
            axis=1, keepdims=True,
        )

        @pl.when(c == N_CHUNKS - 1)
        def _exchange():
            send_buf[:, 0:1] = s_run[...]
            send_buf[:, 1:2] = l_run[...]

            barrier = pltpu.get_barrier_semaphore()
            pl.semaphore_signal(
                barrier, inc=1,
                device_id=partner, device_id_type=pl.DeviceIdType.MESH,
            )
            pl.semaphore_wait(barrier, 1)

            rdma = pltpu.make_async_remote_copy(
                src_ref=send_buf,
                dst_ref=recv_buf,
                send_sem=send_sem,
                recv_sem=recv_sem,
                device_id=partner,
                device_id_type=pl.DeviceIdType.MESH,
            )
            rdma.start()
            rdma.wait()

            s = s_run[...] + recv_buf[:, 0:1]
            lbl = l_run[...] + recv_buf[:, 1:2]
            out_ref[...] = jnp.log(s) - jnp.log(lbl)

    out = pl.pallas_call(
        body,
        grid=(N_CHUNKS,),
        out_shape=jax.ShapeDtypeStruct((T, 1), jnp.float32),
        in_specs=[
            pl.BlockSpec((T, D), lambda c: (0, 0)),
            pl.BlockSpec((D, V_CHUNK), lambda c: (0, c)),
            pl.BlockSpec((T, 1), lambda c: (0, 0)),
        ],
        out_specs=pl.BlockSpec((T, 1), lambda c: (0, 0)),
        scratch_shapes=[
            pltpu.VMEM((T, 1), jnp.float32),
            pltpu.VMEM((T, 1), jnp.float32),
            pltpu.VMEM((T, 8), jnp.float32),
            pltpu.VMEM((T, 8), jnp.float32),
            pltpu.SemaphoreType.DMA,
            pltpu.SemaphoreType.DMA,
        ],
        compiler_params=pltpu.CompilerParams(
            collective_id=0,
            dimension_semantics=("arbitrary",),
        ),
    )(x, W, labels.reshape(T, 1).astype(jnp.int32))
    return out.reshape(T)


# device time: 24376 ns/iter; 1.1692x vs baseline; 1.0500x over previous
import jax
import jax.numpy as jnp
from jax import lax
from jax.experimental import pallas as pl
from jax.experimental.pallas import tpu as pltpu

T = 512
D = 1024
V_LOCAL = 8192
V_NODE = 4096
CH = 1024
N_CH = V_NODE // CH


def kernel(x, W, labels):
    def body(x_ref, w_ref, labels_ref, out_ref,
             w_buf, send_buf, recv_buf, load_sems, send_sems, recv_sems):
        my_x = lax.axis_index("x")
        my_y = lax.axis_index("y")
        my_id = my_x * 2 + my_y
        peers = [
            (my_x, 1 - my_y),
            (1 - my_x, my_y),
            (1 - my_x, 1 - my_y),
        ]
        col0 = my_x * V_NODE

        def dma(i):
            return pltpu.make_async_copy(
                w_ref.at[:, pl.ds(col0 + i * CH, CH)],
                w_buf.at[i % 2],
                load_sems.at[i % 2],
            )

        dma(0).start()
        dma(1).start()

        xv = x_ref[...]
        labels_col = labels_ref[...]
        ones = jnp.ones((CH, 1), jnp.float32)
        base = my_y * V_LOCAL + col0
        s_acc = jnp.zeros((T, 1), jnp.float32)
        l_acc = jnp.zeros((T, 1), jnp.float32)
        for i in range(N_CH):
            dma(i).wait()
            logits = jnp.dot(xv, w_buf[i % 2],
                             preferred_element_type=jnp.float32)
            cols = lax.broadcasted_iota(jnp.int32, (T, CH), 1) \
                + (base + i * CH)
            e = jnp.exp(logits)
            s_acc += jnp.dot(e, ones, preferred_element_type=jnp.float32)
            l_acc += jnp.dot(jnp.where(cols == labels_col, e, 0.0), ones,
                             preferred_element_type=jnp.float32)
            if i + 2 < N_CH:
                dma(i + 2).start()

        send_buf[:, 0:1] = s_acc
        send_buf[:, 1:2] = l_acc

        barrier = pltpu.get_barrier_semaphore()
        for p in peers:
            pl.semaphore_signal(
                barrier, inc=1,
                device_id=p, device_id_type=pl.DeviceIdType.MESH,
            )
        pl.semaphore_wait(barrier, 3)

        rdmas = []
        for k, p in enumerate(peers):
            r = pltpu.make_async_remote_copy(
                src_ref=send_buf,
                dst_ref=recv_buf.at[my_id],
                send_sem=send_sems.at[k],
                recv_sem=recv_sems.at[my_id],
                device_id=p,
                device_id_type=pl.DeviceIdType.MESH,
            )
            r.start()
            rdmas.append(r)

        for p in peers:
            pid = p[0] * 2 + p[1]
            pltpu.make_async_remote_copy(
                src_ref=send_buf,
                dst_ref=recv_buf.at[pid],
                send_sem=send_sems.at[0],
                recv_sem=recv_sems.at[pid],
                device_id=p,
                device_id_type=pl.DeviceIdType.MESH,
            ).wait_recv()
        for r in rdmas:
            r.wait_send()

        s_tot = s_acc
        l_tot = l_acc
        for p in peers:
            pid = p[0] * 2 + p[1]
            s_tot += recv_buf[pid, :, 0:1]
            l_tot += recv_buf[pid, :, 1:2]
        out_ref[...] = jnp.log(s_tot) - jnp.log(l_tot)

    out = pl.pallas_call(
        body,
        out_shape=jax.ShapeDtypeStruct((T, 1), jnp.float32),
        in_specs=[
            pl.BlockSpec(memory_space=pltpu.VMEM),
            pl.BlockSpec(memory_space=pltpu.MemorySpace.HBM),
            pl.BlockSpec(memory_space=pltpu.VMEM),
        ],
        out_specs=pl.BlockSpec(memory_space=pltpu.VMEM),
        scratch_shapes=[
            pltpu.VMEM((2, D, CH), jnp.float32),
            pltpu.VMEM((T, 8), jnp.float32),
            pltpu.VMEM((4, T, 8), jnp.float32),
            pltpu.SemaphoreType.DMA((2,)),
            pltpu.SemaphoreType.DMA((3,)),
            pltpu.SemaphoreType.DMA((4,)),
        ],
        compiler_params=pltpu.CompilerParams(collective_id=0),
    )(x, W, labels.reshape(T, 1).astype(jnp.int32))
    return out.reshape(T)


# device time: 23093 ns/iter; 1.2342x vs baseline; 1.0556x over previous
import jax
import jax.numpy as jnp
from jax import lax
from jax.experimental import pallas as pl
from jax.experimental.pallas import tpu as pltpu

T = 512
D = 1024
V_LOCAL = 8192
V_NODE = 4096
CH = 512
N_CH = V_NODE // CH
NBUF = 4


def kernel(x, W, labels):
    def body(x_ref, w_ref, labels_ref, out_ref,
             w_buf, send_buf, recv_buf, load_sems, send_sems, recv_sems):
        my_x = lax.axis_index("x")
        my_y = lax.axis_index("y")
        my_id = my_x * 2 + my_y
        peers = [
            (my_x, 1 - my_y),
            (1 - my_x, my_y),
            (1 - my_x, 1 - my_y),
        ]
        col0 = my_x * V_NODE

        def dma(i):
            return pltpu.make_async_copy(
                w_ref.at[:, pl.ds(col0 + i * CH, CH)],
                w_buf.at[i % NBUF],
                load_sems.at[i % NBUF],
            )

        for i in range(NBUF):
            dma(i).start()

        xv = x_ref[...]
        labels_col = labels_ref[...]
        base = my_y * V_LOCAL + col0
        cols0 = lax.broadcasted_iota(jnp.int32, (T, CH), 1)
        acc = jnp.zeros((T, CH), jnp.float32)
        lacc = jnp.zeros((T, CH), jnp.float32)
        for i in range(N_CH):
            dma(i).wait()
            logits = jnp.dot(xv, w_buf[i % NBUF],
                             preferred_element_type=jnp.float32)
            e = jnp.exp(logits)
            acc += e
            lacc += jnp.where(cols0 == labels_col - (base + i * CH), e, 0.0)
            if i + NBUF < N_CH:
                dma(i + NBUF).start()
        s_acc = jnp.sum(acc, axis=1, keepdims=True)
        l_acc = jnp.sum(lacc, axis=1, keepdims=True)

        send_buf[:, 0:1] = s_acc
        send_buf[:, 1:2] = l_acc

        barrier = pltpu.get_barrier_semaphore()
        for p in peers:
            pl.semaphore_signal(
                barrier, inc=1,
                device_id=p, device_id_type=pl.DeviceIdType.MESH,
            )
        pl.semaphore_wait(barrier, 3)

        rdmas = []
        for k, p in enumerate(peers):
            r = pltpu.make_async_remote_copy(
                src_ref=send_buf,
                dst_ref=recv_buf.at[my_id],
                send_sem=send_sems.at[k],
                recv_sem=recv_sems.at[my_id],
                device_id=p,
                device_id_type=pl.DeviceIdType.MESH,
            )
            r.start()
            rdmas.append(r)

        s_tot = s_acc
        l_tot = l_acc
        for p in peers:
            pid = p[0] * 2 + p[1]
            pltpu.make_async_remote_copy(
                src_ref=send_buf,
                dst_ref=recv_buf.at[pid],
                send_sem=send_sems.at[0],
                recv_sem=recv_sems.at[pid],
                device_id=p,
                device_id_type=pl.DeviceIdType.MESH,
            ).wait_recv()
            s_tot += recv_buf[pid, :, 0:1]
            l_tot += recv_buf[pid, :, 1:2]
        for r in rdmas:
            r.wait_send()

        out_ref[...] = jnp.log(s_tot) - jnp.log(l_tot)

    out = pl.pallas_call(
        body,
        out_shape=jax.ShapeDtypeStruct((T, 1), jnp.float32),
        in_specs=[
            pl.BlockSpec(memory_space=pltpu.MemorySpace.VMEM),
            pl.BlockSpec(memory_space=pltpu.MemorySpace.HBM),
            pl.BlockSpec(memory_space=pltpu.MemorySpace.VMEM),
        ],
        out_specs=pl.BlockSpec(memory_space=pltpu.MemorySpace.VMEM),
        scratch_shapes=[
            pltpu.VMEM((NBUF, D, CH), jnp.float32),
            pltpu.VMEM((T, 8), jnp.float32),
            pltpu.VMEM((4, T, 8), jnp.float32),
            pltpu.SemaphoreType.DMA((NBUF,)),
            pltpu.SemaphoreType.DMA((3,)),
            pltpu.SemaphoreType.DMA((4,)),
        ],
        compiler_params=pltpu.CompilerParams(collective_id=0),
    )(x, W, labels.reshape(T, 1).astype(jnp.int32))
    return out.reshape(T)


# device time: 22732 ns/iter; 1.2538x vs baseline; 1.0159x over previous
import jax
import jax.numpy as jnp
from jax import lax
from jax.experimental import pallas as pl
from jax.experimental.pallas import tpu as pltpu

T = 512
D = 1024
V_LOCAL = 8192
V_NODE = 4096
CH = 512
N_CH = V_NODE // CH
NBUF = 4


def kernel(x, W, labels):
    def body(x_ref, w_ref, labels_ref, out_ref,
             w_buf, send_buf, recv_buf, load_sems, send_sems, recv_sems):
        my_x = lax.axis_index("x")
        my_y = lax.axis_index("y")
        my_id = my_x * 2 + my_y
        peers = [
            (my_x, 1 - my_y),
            (1 - my_x, my_y),
            (1 - my_x, 1 - my_y),
        ]
        col0 = my_x * V_NODE

        barrier = pltpu.get_barrier_semaphore()
        for p in peers:
            pl.semaphore_signal(
                barrier, inc=1,
                device_id=p, device_id_type=pl.DeviceIdType.MESH,
            )

        def dma(i):
            return pltpu.make_async_copy(
                w_ref.at[:, pl.ds(col0 + i * CH, CH)],
                w_buf.at[i % NBUF],
                load_sems.at[i % NBUF],
            )

        for i in range(NBUF):
            dma(i).start()

        xv = x_ref[...]
        labels_col = labels_ref[...]
        base = my_y * V_LOCAL + col0
        cols0 = lax.broadcasted_iota(jnp.int32, (T, CH), 1)
        acc = jnp.zeros((T, CH), jnp.float32)
        lacc = jnp.zeros((T, CH), jnp.float32)
        for i in range(N_CH):
            dma(i).wait()
            logits = jnp.dot(xv, w_buf[i % NBUF],
                             preferred_element_type=jnp.float32)
            e = jnp.exp(logits)
            acc += e
            lacc += jnp.where(cols0 == labels_col - (base + i * CH), e, 0.0)
            if i + NBUF < N_CH:
                dma(i + NBUF).start()
        s_acc = jnp.sum(acc, axis=1, keepdims=True)
        l_acc = jnp.sum(lacc, axis=1, keepdims=True)

        send_buf[:, 0:1] = s_acc
        send_buf[:, 1:2] = l_acc

        pl.semaphore_wait(barrier, 3)

        rdmas = []
        for k, p in enumerate(peers):
            r = pltpu.make_async_remote_copy(
                src_ref=send_buf,
                dst_ref=recv_buf.at[my_id],
                send_sem=send_sems.at[k],
                recv_sem=recv_sems.at[my_id],
                device_id=p,
                device_id_type=pl.DeviceIdType.MESH,
            )
            r.start()
            rdmas.append(r)

        s_tot = s_acc
        l_tot = l_acc
        for p in peers:
            pid = p[0] * 2 + p[1]
            pltpu.make_async_remote_copy(
                src_ref=send_buf,
                dst_ref=recv_buf.at[pid],
                send_sem=send_sems.at[0],
                recv_sem=recv_sems.at[pid],
                device_id=p,
                device_id_type=pl.DeviceIdType.MESH,
            ).wait_recv()
            s_tot += recv_buf[pid, :, 0:1]
            l_tot += recv_buf[pid, :, 1:2]
        for r in rdmas:
            r.wait_send()

        out_ref[...] = jnp.log(s_tot) - jnp.log(l_tot)

    out = pl.pallas_call(
        body,
        out_shape=jax.ShapeDtypeStruct((T, 1), jnp.float32),
        in_specs=[
            pl.BlockSpec(memory_space=pltpu.MemorySpace.VMEM),
            pl.BlockSpec(memory_space=pltpu.MemorySpace.HBM),
            pl.BlockSpec(memory_space=pltpu.MemorySpace.VMEM),
        ],
        out_specs=pl.BlockSpec(memory_space=pltpu.MemorySpace.VMEM),
        scratch_shapes=[
            pltpu.VMEM((NBUF, D, CH), jnp.float32),
            pltpu.VMEM((T, 8), jnp.float32),
            pltpu.VMEM((4, T, 8), jnp.float32),
            pltpu.SemaphoreType.DMA((NBUF,)),
            pltpu.SemaphoreType.DMA((3,)),
            pltpu.SemaphoreType.DMA((4,)),
        ],
        compiler_params=pltpu.CompilerParams(collective_id=0),
    )(x, W, labels.reshape(T, 1).astype(jnp.int32))
    return out.reshape(T)


# device time: 22353 ns/iter; 1.2750x vs baseline; 1.0170x over previous
import jax
import jax.numpy as jnp
from jax import lax
from jax.experimental import pallas as pl
from jax.experimental.pallas import tpu as pltpu

T = 512
D = 1024
V_LOCAL = 8192
V_NODE = 4096
CH = 1024
N_CH = V_NODE // CH
NBUF = 4


def kernel(x, W, labels):
    def body(x_ref, w_ref, labels_ref, out_ref,
             w_buf, send_buf, recv_buf, load_sems, send_sems, recv_sems):
        my_x = lax.axis_index("x")
        my_y = lax.axis_index("y")
        my_id = my_x * 2 + my_y
        peers = [
            (my_x, 1 - my_y),
            (1 - my_x, my_y),
            (1 - my_x, 1 - my_y),
        ]
        col0 = my_x * V_NODE

        barrier = pltpu.get_barrier_semaphore()
        for p in peers:
            pl.semaphore_signal(
                barrier, inc=1,
                device_id=p, device_id_type=pl.DeviceIdType.MESH,
            )

        def dma(i):
            return pltpu.make_async_copy(
                w_ref.at[:, pl.ds(col0 + i * CH, CH)],
                w_buf.at[i % NBUF],
                load_sems.at[i % NBUF],
            )

        for i in range(NBUF):
            dma(i).start()

        xv = x_ref[...]
        labels_col = labels_ref[...]
        base = my_y * V_LOCAL + col0
        cols0 = lax.broadcasted_iota(jnp.int32, (T, CH), 1)
        acc = jnp.zeros((T, CH), jnp.float32)
        lacc = jnp.zeros((T, CH), jnp.float32)
        for i in range(N_CH):
            dma(i).wait()
            logits = jnp.dot(xv, w_buf[i % NBUF],
                             preferred_element_type=jnp.float32)
            e = jnp.exp(logits)
            acc += e
            lacc += jnp.where(cols0 == labels_col - (base + i * CH), e, 0.0)
            if i + NBUF < N_CH:
                dma(i + NBUF).start()
        s_acc = jnp.sum(acc, axis=1, keepdims=True)
        l_acc = jnp.sum(lacc, axis=1, keepdims=True)

        send_buf[:, 0:1] = s_acc
        send_buf[:, 1:2] = l_acc

        pl.semaphore_wait(barrier, 3)

        rdmas = []
        for k, p in enumerate(peers):
            r = pltpu.make_async_remote_copy(
                src_ref=send_buf,
                dst_ref=recv_buf.at[my_id],
                send_sem=send_sems.at[k],
                recv_sem=recv_sems.at[my_id],
                device_id=p,
                device_id_type=pl.DeviceIdType.MESH,
            )
            r.start()
            rdmas.append(r)

        s_tot = s_acc
        l_tot = l_acc
        for p in peers:
            pid = p[0] * 2 + p[1]
            pltpu.make_async_remote_copy(
                src_ref=send_buf,
                dst_ref=recv_buf.at[pid],
                send_sem=send_sems.at[0],
                recv_sem=recv_sems.at[pid],
                device_id=p,
                device_id_type=pl.DeviceIdType.MESH,
            ).wait_recv()
            s_tot += recv_buf[pid, :, 0:1]
            l_tot += recv_buf[pid, :, 1:2]
        for r in rdmas:
            r.wait_send()

        out_ref[...] = jnp.log(s_tot) - jnp.log(l_tot)

    out = pl.pallas_call(
        body,
        out_shape=jax.ShapeDtypeStruct((T, 1), jnp.float32),
        in_specs=[
            pl.BlockSpec(memory_space=pltpu.MemorySpace.VMEM),
            pl.BlockSpec(memory_space=pltpu.MemorySpace.HBM),
            pl.BlockSpec(memory_space=pltpu.MemorySpace.VMEM),
        ],
        out_specs=pl.BlockSpec(memory_space=pltpu.MemorySpace.VMEM),
        scratch_shapes=[
            pltpu.VMEM((NBUF, D, CH), jnp.float32),
            pltpu.VMEM((T, 8), jnp.float32),
            pltpu.VMEM((4, T, 8), jnp.float32),
            pltpu.SemaphoreType.DMA((NBUF,)),
            pltpu.SemaphoreType.DMA((3,)),
            pltpu.SemaphoreType.DMA((4,)),
        ],
        compiler_params=pltpu.CompilerParams(collective_id=0),
    )(x, W, labels.reshape(T, 1).astype(jnp.int32))
    return out.reshape(T)
